# Initial kernel scaffold; baseline (speedup 1.0000x reference)
import jax
import jax.numpy as jnp
from jax.experimental import pallas as pl


def kernel(x, edge_index, W_root, W_neigh, b, gamma, beta):
    raise NotImplementedError("write your pallas kernel here")


# speedup vs baseline: 3.1708x; 3.1708x over previous
import functools

import jax
import jax.numpy as jnp
from jax import lax
from jax.experimental import pallas as pl
from jax.experimental.pallas import tpu as pltpu
from jax.experimental.pallas import tpu_sc as plsc

N = 10000
D = 128
E = 320000
L = 4

NC = 2
NS = 16
NW = NC * NS
B = 128
CH = 80
EPT = CH * B
E_PAD = EPT * NW
N_ACC = 10240
RPT = N_ACC // NS
BN = 1000

_mesh = plsc.VectorSubcoreMesh(core_axis_name="c", subcore_axis_name="s")


def _agg_body(h_hbm, src_hbm, dst_hbm, z_hbm, p_hbm, src_v, dst_v, rows_v,
              acc_sh, sem):
    c = lax.axis_index("c")
    s = lax.axis_index("s")
    wid = c * NS + s
    pltpu.sync_copy(z_hbm, acc_sh.at[pl.ds(s * RPT, RPT)])
    pltpu.sync_copy(src_hbm.at[pl.ds(wid * CH, CH)], src_v)
    pltpu.sync_copy(dst_hbm.at[pl.ds(wid * CH, CH)], dst_v)
    plsc.subcore_barrier()

    def chunk(j, carry):
        pltpu.async_copy(h_hbm.at[src_v.at[j]], rows_v, sem).wait()
        pltpu.sync_copy(rows_v, acc_sh.at[dst_v.at[j]], add=True)
        return carry

    lax.fori_loop(0, CH, chunk, 0)
    plsc.subcore_barrier()
    pltpu.sync_copy(acc_sh.at[pl.ds(s * RPT, RPT)],
                    p_hbm.at[pl.ds(c * N_ACC + s * RPT, RPT)])


_agg = pl.kernel(
    _agg_body,
    out_type=jax.ShapeDtypeStruct((NC * N_ACC, D), jnp.float32),
    mesh=_mesh,
    scratch_types=[
        pltpu.VMEM((CH, B), jnp.int32),
        pltpu.VMEM((CH, B), jnp.int32),
        pltpu.VMEM((B, D), jnp.float32),
        pltpu.VMEM_SHARED((N_ACC, D), jnp.float32),
        pltpu.SemaphoreType.DMA,
    ],
)


def _deg_body(dst_hbm, ones_hbm, z_hbm, pd_hbm, dst_v, ones_v, acc_sh):
    c = lax.axis_index("c")
    s = lax.axis_index("s")
    wid = c * NS + s
    pltpu.sync_copy(z_hbm, acc_sh.at[pl.ds(s * RPT, RPT)])
    pltpu.sync_copy(ones_hbm, ones_v)
    pltpu.sync_copy(dst_hbm.at[pl.ds(wid * CH, CH)], dst_v)
    plsc.subcore_barrier()

    def chunk(j, carry):
        pltpu.sync_copy(ones_v, acc_sh.at[dst_v.at[j]], add=True)
        return carry

    lax.fori_loop(0, CH, chunk, 0)
    plsc.subcore_barrier()
    pltpu.sync_copy(acc_sh.at[pl.ds(s * RPT, RPT)],
                    pd_hbm.at[pl.ds(c * N_ACC + s * RPT, RPT)])


_deg = pl.kernel(
    _deg_body,
    out_type=jax.ShapeDtypeStruct((NC * N_ACC, D), jnp.float32),
    mesh=_mesh,
    scratch_types=[
        pltpu.VMEM((CH, B), jnp.int32),
        pltpu.VMEM((B, D), jnp.float32),
        pltpu.VMEM_SHARED((N_ACC, D), jnp.float32),
    ],
)


def _inv_body(d0_ref, d1_ref, o_ref):
    o_ref[...] = 1.0 / jnp.maximum(d0_ref[...] + d1_ref[...], 1.0)


_inv = pl.pallas_call(
    _inv_body,
    out_shape=jax.ShapeDtypeStruct((N, D), jnp.float32),
    grid=(N // BN,),
    in_specs=[
        pl.BlockSpec((BN, D), lambda i: (i, 0)),
        pl.BlockSpec((BN, D), lambda i: (i, 0)),
    ],
    out_specs=pl.BlockSpec((BN, D), lambda i: (i, 0)),
)


def _dense_body(h_ref, p0_ref, p1_ref, inv_ref, wr_ref, wn_ref, b_ref, g_ref,
                be_ref, o_ref):
    h = h_ref[...]
    agg = (p0_ref[...] + p1_ref[...]) * inv_ref[...]
    out = jnp.dot(h, wr_ref[...], preferred_element_type=jnp.float32)
    out = out + jnp.dot(agg, wn_ref[...], preferred_element_type=jnp.float32)
    out = out + b_ref[...]
    mu = jnp.mean(out, axis=-1, keepdims=True)
    var = jnp.mean((out - mu) ** 2, axis=-1, keepdims=True)
    out = (out - mu) * lax.rsqrt(var + 1e-5) * g_ref[...] + be_ref[...]
    o_ref[...] = h + jnp.maximum(out, 0.0)


_dense = pl.pallas_call(
    _dense_body,
    out_shape=jax.ShapeDtypeStruct((N, D), jnp.float32),
    grid=(N // BN,),
    in_specs=[
        pl.BlockSpec((BN, D), lambda i: (i, 0)),
        pl.BlockSpec((BN, D), lambda i: (i, 0)),
        pl.BlockSpec((BN, D), lambda i: (i, 0)),
        pl.BlockSpec((BN, D), lambda i: (i, 0)),
        pl.BlockSpec((D, D), lambda i: (0, 0)),
        pl.BlockSpec((D, D), lambda i: (0, 0)),
        pl.BlockSpec((1, D), lambda i: (0, 0)),
        pl.BlockSpec((1, D), lambda i: (0, 0)),
        pl.BlockSpec((1, D), lambda i: (0, 0)),
    ],
    out_specs=pl.BlockSpec((BN, D), lambda i: (i, 0)),
)


def kernel(x, edge_index, W_root, W_neigh, b, gamma, beta):
    src = edge_index[0]
    dst = edge_index[1]
    pad_e = E_PAD - E
    src_p = jnp.concatenate(
        [src, jnp.zeros((pad_e,), jnp.int32)]).reshape(NW * CH, B)
    dst_p = jnp.concatenate(
        [dst, jnp.full((pad_e,), N, jnp.int32)]).reshape(NW * CH, B)
    zeros = jnp.zeros((RPT, D), jnp.float32)
    ones = jnp.ones((B, D), jnp.float32)

    pd = _deg(dst_p, ones, zeros)
    inv = _inv(pd[:N], pd[N_ACC:N_ACC + N])

    h = x
    for i in range(L):
        p = _agg(h, src_p, dst_p, zeros)
        h = _dense(h, p[:N], p[N_ACC:N_ACC + N], inv,
                   W_root[i], W_neigh[i],
                   b[i].reshape(1, D), gamma[i].reshape(1, D),
                   beta[i].reshape(1, D))
    return h


# speedup vs baseline: 3.5671x; 1.1250x over previous
import functools

import jax
import jax.numpy as jnp
from jax import lax
from jax.experimental import pallas as pl
from jax.experimental.pallas import tpu as pltpu
from jax.experimental.pallas import tpu_sc as plsc

N = 10000
D = 128
E = 320000
L = 4

NC = 2
NS = 16
NW = NC * NS
B = 128
CH = 80
HCH = CH // 2
EPT = CH * B
E_PAD = EPT * NW
N_ACC = 10240
RPT = N_ACC // NS
BN = 1000

_mesh = plsc.VectorSubcoreMesh(core_axis_name="c", subcore_axis_name="s")


def _agg_body(h_hbm, src_hbm, dst_hbm, z_hbm, p_hbm, src_v, dst_v, rows0,
              rows1, acc_sh, gsem0, gsem1):
    c = lax.axis_index("c")
    s = lax.axis_index("s")
    wid = c * NS + s
    pltpu.sync_copy(z_hbm, acc_sh.at[pl.ds(s * RPT, RPT)])
    plsc.subcore_barrier()

    for half in range(2):
        base = wid * CH + half * HCH
        pltpu.sync_copy(src_hbm.at[pl.ds(base, HCH)], src_v)
        pltpu.sync_copy(dst_hbm.at[pl.ds(base, HCH)], dst_v)
        pltpu.async_copy(h_hbm.at[src_v.at[0]], rows0, gsem0)

        def pair(i, carry):
            j = 2 * i
            pltpu.async_copy(h_hbm.at[src_v.at[j + 1]], rows1, gsem1)
            pltpu.make_async_copy(h_hbm.at[src_v.at[j]], rows0, gsem0).wait()
            pltpu.sync_copy(rows0, acc_sh.at[dst_v.at[j]], add=True)

            @pl.when(j + 2 < HCH)
            def _():
                pltpu.async_copy(h_hbm.at[src_v.at[j + 2]], rows0, gsem0)

            pltpu.make_async_copy(h_hbm.at[src_v.at[j + 1]], rows1,
                                  gsem1).wait()
            pltpu.sync_copy(rows1, acc_sh.at[dst_v.at[j + 1]], add=True)
            return carry

        lax.fori_loop(0, HCH // 2, pair, 0)
    plsc.subcore_barrier()
    pltpu.sync_copy(acc_sh.at[pl.ds(s * RPT, RPT)],
                    p_hbm.at[pl.ds(c * N_ACC + s * RPT, RPT)])


_agg = pl.kernel(
    _agg_body,
    out_type=jax.ShapeDtypeStruct((NC * N_ACC, D), jnp.float32),
    mesh=_mesh,
    scratch_types=[
        pltpu.VMEM((HCH, B), jnp.int32),
        pltpu.VMEM((HCH, B), jnp.int32),
        pltpu.VMEM((B, D), jnp.float32),
        pltpu.VMEM((B, D), jnp.float32),
        pltpu.VMEM_SHARED((N_ACC, D), jnp.float32),
        pltpu.SemaphoreType.DMA,
        pltpu.SemaphoreType.DMA,
    ],
)


def _deg_body(dst_hbm, ones_hbm, z_hbm, pd_hbm, dst_v, ones_v, acc_sh):
    c = lax.axis_index("c")
    s = lax.axis_index("s")
    wid = c * NS + s
    pltpu.sync_copy(z_hbm, acc_sh.at[pl.ds(s * RPT, RPT)])
    pltpu.sync_copy(ones_hbm, ones_v)
    pltpu.sync_copy(dst_hbm.at[pl.ds(wid * CH, CH)], dst_v)
    plsc.subcore_barrier()

    def chunk(j, carry):
        pltpu.sync_copy(ones_v, acc_sh.at[dst_v.at[j]], add=True)
        return carry

    lax.fori_loop(0, CH, chunk, 0)
    plsc.subcore_barrier()
    pltpu.sync_copy(acc_sh.at[pl.ds(s * RPT, RPT)],
                    pd_hbm.at[pl.ds(c * N_ACC + s * RPT, RPT)])


_deg = pl.kernel(
    _deg_body,
    out_type=jax.ShapeDtypeStruct((NC * N_ACC, D), jnp.float32),
    mesh=_mesh,
    scratch_types=[
        pltpu.VMEM((CH, B), jnp.int32),
        pltpu.VMEM((B, D), jnp.float32),
        pltpu.VMEM_SHARED((N_ACC, D), jnp.float32),
    ],
)


def _inv_body(d0_ref, d1_ref, o_ref):
    o_ref[...] = 1.0 / jnp.maximum(d0_ref[...] + d1_ref[...], 1.0)


_inv = pl.pallas_call(
    _inv_body,
    out_shape=jax.ShapeDtypeStruct((N, D), jnp.float32),
    grid=(N // BN,),
    in_specs=[
        pl.BlockSpec((BN, D), lambda i: (i, 0)),
        pl.BlockSpec((BN, D), lambda i: (i, 0)),
    ],
    out_specs=pl.BlockSpec((BN, D), lambda i: (i, 0)),
)


def _dense_body(h_ref, p0_ref, p1_ref, inv_ref, wr_ref, wn_ref, b_ref, g_ref,
                be_ref, o_ref):
    h = h_ref[...]
    agg = (p0_ref[...] + p1_ref[...]) * inv_ref[...]
    out = jnp.dot(h, wr_ref[...], preferred_element_type=jnp.float32)
    out = out + jnp.dot(agg, wn_ref[...], preferred_element_type=jnp.float32)
    out = out + b_ref[...]
    mu = jnp.mean(out, axis=-1, keepdims=True)
    var = jnp.mean((out - mu) ** 2, axis=-1, keepdims=True)
    out = (out - mu) * lax.rsqrt(var + 1e-5) * g_ref[...] + be_ref[...]
    o_ref[...] = h + jnp.maximum(out, 0.0)


_dense = pl.pallas_call(
    _dense_body,
    out_shape=jax.ShapeDtypeStruct((N, D), jnp.float32),
    grid=(N // BN,),
    in_specs=[
        pl.BlockSpec((BN, D), lambda i: (i, 0)),
        pl.BlockSpec((BN, D), lambda i: (i, 0)),
        pl.BlockSpec((BN, D), lambda i: (i, 0)),
        pl.BlockSpec((BN, D), lambda i: (i, 0)),
        pl.BlockSpec((D, D), lambda i: (0, 0)),
        pl.BlockSpec((D, D), lambda i: (0, 0)),
        pl.BlockSpec((1, D), lambda i: (0, 0)),
        pl.BlockSpec((1, D), lambda i: (0, 0)),
        pl.BlockSpec((1, D), lambda i: (0, 0)),
    ],
    out_specs=pl.BlockSpec((BN, D), lambda i: (i, 0)),
)


def kernel(x, edge_index, W_root, W_neigh, b, gamma, beta):
    src = edge_index[0]
    dst = edge_index[1]
    pad_e = E_PAD - E
    src_p = jnp.concatenate(
        [src, jnp.zeros((pad_e,), jnp.int32)]).reshape(NW * CH, B)
    dst_p = jnp.concatenate(
        [dst, jnp.full((pad_e,), N, jnp.int32)]).reshape(NW * CH, B)
    zeros = jnp.zeros((RPT, D), jnp.float32)
    ones = jnp.ones((B, D), jnp.float32)

    pd = _deg(dst_p, ones, zeros)
    inv = _inv(pd[:N], pd[N_ACC:N_ACC + N])

    h = x
    for i in range(L):
        p = _agg(h, src_p, dst_p, zeros)
        h = _dense(h, p[:N], p[N_ACC:N_ACC + N], inv,
                   W_root[i], W_neigh[i],
                   b[i].reshape(1, D), gamma[i].reshape(1, D),
                   beta[i].reshape(1, D))
    return h
